# baseline (speedup 1.0000x reference)
import dataclasses
import functools

import jax
import jax.numpy as jnp
from jax import lax
from jax.experimental import pallas as pl
from jax.experimental.pallas import tpu as pltpu
from jax.experimental.pallas import tpu_sc as plsc

_N_CATS = 26
_VOCAB = 100000
_VPAD = 100352
_N_EMBED = 16
_N_NUM = 13
_N_FEAT = _N_CATS + _N_NUM
_BATCH = 16384

_NC = 2
_NS = 16
_NW = _NC * _NS
_SPW = _BATCH // _NW
_CHUNK = 512
_NCHUNK = _SPW // _CHUNK

_PIECES = ((0, 14), (14, 12))
_TPB = 2


def _indices(x_t):
    starts = {}
    for t0, tn in _PIECES:
        for t in range(t0, t0 + tn):
            starts[t] = t0

    def body(x_ref, o_ref):
        for t in range(_N_CATS):
            rel = t - starts[t]
            o_ref[pl.ds(t * _BATCH, _BATCH)] = (
                x_ref[t].astype(jnp.int32) + rel * _VPAD)

    return pl.pallas_call(
        body,
        grid=(1,),
        in_specs=[pl.BlockSpec((32, _BATCH), lambda i: (0, 0))],
        out_specs=pl.BlockSpec((_N_CATS * _BATCH,), lambda i: (0,)),
        out_shape=jax.ShapeDtypeStruct((_N_CATS * _BATCH,), jnp.int32),
    )(x_t)


def _project(tab_t, w_col, t0, tn):
    def body(tab_ref, w_ref, p_ref):
        for u in range(_TPB):
            p_ref[pl.ds(u * _VPAD, _VPAD)] = jnp.sum(
                tab_ref[u] * w_ref[u], axis=0)

    return pl.pallas_call(
        body,
        grid=(tn // _TPB,),
        in_specs=[
            pl.BlockSpec((_TPB, _N_EMBED, _VPAD),
                         lambda t: (t0 // _TPB + t, 0, 0)),
            pl.BlockSpec((_TPB, _N_EMBED, 1), lambda t: (t0 // _TPB + t, 0, 0)),
        ],
        out_specs=pl.BlockSpec((_TPB * _VPAD,), lambda t: (t,)),
        out_shape=jax.ShapeDtypeStruct((tn * _VPAD,), jnp.float32),
    )(tab_t, w_col)


def _sc_compiler_params():
    cp = pltpu.CompilerParams(use_tc_tiling_on_sc=False)
    if "needs_layout_passes" in pltpu.CompilerParams.__dataclass_fields__:
        cp = dataclasses.replace(cp, needs_layout_passes=False)
    return cp


def _sc_gather_sum(xi_flat, p_flat, t0, tn):
    mesh = plsc.VectorSubcoreMesh(core_axis_name="c", subcore_axis_name="s")

    @functools.partial(
        pl.kernel,
        out_type=jax.ShapeDtypeStruct((_BATCH,), jnp.float32),
        mesh=mesh,
        scratch_types=[
            pltpu.VMEM((tn * _SPW,), jnp.int32),
            pltpu.VMEM((tn * _CHUNK,), jnp.float32),
            pltpu.VMEM((_CHUNK,), jnp.float32),
            pltpu.SemaphoreType.DMA,
            pltpu.SemaphoreType.DMA,
        ],
        compiler_params=_sc_compiler_params(),
    )
    def k(xi_hbm, p_hbm, out_hbm, idx_v, vals_v, sum_v, isem, gsem):
        wid = lax.axis_index("s") * _NC + lax.axis_index("c")
        base = wid * _SPW

        idx_copies = [
            pltpu.async_copy(
                xi_hbm.at[pl.ds(
                    pl.multiple_of((t0 + j) * _BATCH + base, _SPW), _SPW)],
                idx_v.at[pl.ds(j * _SPW, _SPW)],
                isem,
            )
            for j in range(tn)
        ]
        for cp in idx_copies:
            cp.wait()

        @pl.loop(0, _NCHUNK)
        def _chunk(c):
            coff = c * _CHUNK

            gathers = [
                pltpu.async_copy(
                    p_hbm.at[idx_v.at[pl.ds(
                        pl.multiple_of(j * _SPW + coff, _CHUNK), _CHUNK)]],
                    vals_v.at[pl.ds(j * _CHUNK, _CHUNK)],
                    gsem,
                )
                for j in range(tn)
            ]
            for cp in gathers:
                cp.wait()

            for g in range(_CHUNK // 16):
                acc = vals_v[pl.ds(g * 16, 16)]
                for j in range(1, tn):
                    acc = acc + vals_v[pl.ds(j * _CHUNK + g * 16, 16)]
                sum_v[pl.ds(g * 16, 16)] = acc

            pltpu.sync_copy(
                sum_v,
                out_hbm.at[pl.ds(pl.multiple_of(base + coff, _CHUNK), _CHUNK)])

    return k(xi_flat, p_flat)


_BLK = 4096


def _numeric(x_t, w_full_t, bias):
    def body(x_ref, wn_ref, b_ref, o_ref):
        xb = x_ref[...].astype(jnp.bfloat16).astype(jnp.float32)
        wb = wn_ref[...].astype(jnp.bfloat16).astype(jnp.float32)
        o_ref[...] = jnp.sum(xb * wb, axis=0) + b_ref[0, 0]

    return pl.pallas_call(
        body,
        grid=(_BATCH // _BLK,),
        in_specs=[
            pl.BlockSpec((_N_FEAT, _BLK), lambda i: (0, i)),
            pl.BlockSpec((_N_FEAT, 1), lambda i: (0, 0)),
            pl.BlockSpec((1, 1), lambda i: (0, 0)),
        ],
        out_specs=pl.BlockSpec((_BLK,), lambda i: (i,)),
        out_shape=jax.ShapeDtypeStruct((_BATCH,), jnp.float32),
    )(x_t, w_full_t, bias)


def _combine(parts):
    def body(*refs):
        o_ref = refs[-1]
        acc = refs[0][...]
        for r in refs[1:-1]:
            acc = acc + r[...]
        o_ref[...] = acc

    return pl.pallas_call(
        body,
        grid=(_BATCH // _BLK,),
        in_specs=[pl.BlockSpec((_BLK,), lambda i: (i,))] * len(parts),
        out_specs=pl.BlockSpec((_BLK,), lambda i: (i,)),
        out_shape=jax.ShapeDtypeStruct((_BATCH,), jnp.float32),
    )(*parts)


def kernel(x, tables, W, b):
    x_t = jnp.transpose(x, (1, 0))
    tab_t = jnp.transpose(tables, (0, 2, 1))
    w_col = W[0, : _N_CATS * _N_EMBED].reshape(_N_CATS, _N_EMBED)[:, :, None]
    w_full_t = jnp.concatenate(
        [jnp.zeros((_N_CATS,), jnp.float32), W[0, _N_CATS * _N_EMBED:]]
    ).reshape(_N_FEAT, 1)
    bias = b.reshape(1, 1)

    xi_flat = _indices(x_t)
    embs = []
    for t0, tn in _PIECES:
        p_i = _project(tab_t, w_col, t0, tn)
        embs.append(_sc_gather_sum(xi_flat, p_i, t0, tn))
    num = _numeric(x_t, w_full_t, bias)
    return _combine(embs + [num]).reshape(_BATCH, 1)


# speedup vs baseline: 1.0125x; 1.0125x over previous
import dataclasses
import functools

import jax
import jax.numpy as jnp
from jax import lax
from jax.experimental import pallas as pl
from jax.experimental.pallas import tpu as pltpu
from jax.experimental.pallas import tpu_sc as plsc

_N_CATS = 26
_VOCAB = 100000
_VPAD = 100352
_N_EMBED = 16
_N_NUM = 13
_N_FEAT = _N_CATS + _N_NUM
_BATCH = 16384

_NC = 2
_NS = 16
_NW = _NC * _NS
_SPW = _BATCH // _NW
_CHUNK = 512
_NCHUNK = _SPW // _CHUNK

_PIECES = ((0, 14), (14, 12))
_TPB = 2


def _project_first(tab_t, w_col, x_t, t0, tn):
    starts = {}
    for p0, pn in _PIECES:
        for t in range(p0, p0 + pn):
            starts[t] = p0

    def body(tab_ref, w_ref, x_ref, p_ref, xi_ref):
        for u in range(_TPB):
            p_ref[pl.ds(u * _VPAD, _VPAD)] = jnp.sum(
                tab_ref[u] * w_ref[u], axis=0)

        @pl.when(pl.program_id(0) == 0)
        def _():
            for t in range(_N_CATS):
                xi_ref[pl.ds(t * _BATCH, _BATCH)] = (
                    x_ref[t].astype(jnp.int32) + (t - starts[t]) * _VPAD)

    return pl.pallas_call(
        body,
        grid=(tn // _TPB,),
        in_specs=[
            pl.BlockSpec((_TPB, _N_EMBED, _VPAD),
                         lambda t: (t0 // _TPB + t, 0, 0)),
            pl.BlockSpec((_TPB, _N_EMBED, 1), lambda t: (t0 // _TPB + t, 0, 0)),
            pl.BlockSpec((32, _BATCH), lambda t: (0, 0)),
        ],
        out_specs=[
            pl.BlockSpec((_TPB * _VPAD,), lambda t: (t,)),
            pl.BlockSpec((_N_CATS * _BATCH,), lambda t: (0,)),
        ],
        out_shape=[
            jax.ShapeDtypeStruct((tn * _VPAD,), jnp.float32),
            jax.ShapeDtypeStruct((_N_CATS * _BATCH,), jnp.int32),
        ],
    )(tab_t, w_col, x_t)


def _project(tab_t, w_col, t0, tn):
    def body(tab_ref, w_ref, p_ref):
        for u in range(_TPB):
            p_ref[pl.ds(u * _VPAD, _VPAD)] = jnp.sum(
                tab_ref[u] * w_ref[u], axis=0)

    return pl.pallas_call(
        body,
        grid=(tn // _TPB,),
        in_specs=[
            pl.BlockSpec((_TPB, _N_EMBED, _VPAD),
                         lambda t: (t0 // _TPB + t, 0, 0)),
            pl.BlockSpec((_TPB, _N_EMBED, 1), lambda t: (t0 // _TPB + t, 0, 0)),
        ],
        out_specs=pl.BlockSpec((_TPB * _VPAD,), lambda t: (t,)),
        out_shape=jax.ShapeDtypeStruct((tn * _VPAD,), jnp.float32),
    )(tab_t, w_col)


def _sc_compiler_params():
    cp = pltpu.CompilerParams(use_tc_tiling_on_sc=False)
    if "needs_layout_passes" in pltpu.CompilerParams.__dataclass_fields__:
        cp = dataclasses.replace(cp, needs_layout_passes=False)
    return cp


def _sc_gather_sum(xi_flat, p_flat, t0, tn):
    mesh = plsc.VectorSubcoreMesh(core_axis_name="c", subcore_axis_name="s")

    @functools.partial(
        pl.kernel,
        out_type=jax.ShapeDtypeStruct((_BATCH,), jnp.float32),
        mesh=mesh,
        scratch_types=[
            pltpu.VMEM((tn * _SPW,), jnp.int32),
            pltpu.VMEM((tn * _CHUNK,), jnp.float32),
            pltpu.VMEM((_CHUNK,), jnp.float32),
            pltpu.SemaphoreType.DMA,
            pltpu.SemaphoreType.DMA,
        ],
        compiler_params=_sc_compiler_params(),
    )
    def k(xi_hbm, p_hbm, out_hbm, idx_v, vals_v, sum_v, isem, gsem):
        wid = lax.axis_index("s") * _NC + lax.axis_index("c")
        base = wid * _SPW

        idx_copies = [
            pltpu.async_copy(
                xi_hbm.at[pl.ds(
                    pl.multiple_of((t0 + j) * _BATCH + base, _SPW), _SPW)],
                idx_v.at[pl.ds(j * _SPW, _SPW)],
                isem,
            )
            for j in range(tn)
        ]
        for cp in idx_copies:
            cp.wait()

        @pl.loop(0, _NCHUNK)
        def _chunk(c):
            coff = c * _CHUNK

            gathers = [
                pltpu.async_copy(
                    p_hbm.at[idx_v.at[pl.ds(
                        pl.multiple_of(j * _SPW + coff, _CHUNK), _CHUNK)]],
                    vals_v.at[pl.ds(j * _CHUNK, _CHUNK)],
                    gsem,
                )
                for j in range(tn)
            ]
            for cp in gathers:
                cp.wait()

            for g in range(_CHUNK // 16):
                acc = vals_v[pl.ds(g * 16, 16)]
                for j in range(1, tn):
                    acc = acc + vals_v[pl.ds(j * _CHUNK + g * 16, 16)]
                sum_v[pl.ds(g * 16, 16)] = acc

            pltpu.sync_copy(
                sum_v,
                out_hbm.at[pl.ds(pl.multiple_of(base + coff, _CHUNK), _CHUNK)])

    return k(xi_flat, p_flat)


_BLK = 4096


def _numeric(x_t, w_full_t, bias):
    def body(x_ref, wn_ref, b_ref, o_ref):
        xb = x_ref[...].astype(jnp.bfloat16).astype(jnp.float32)
        wb = wn_ref[...].astype(jnp.bfloat16).astype(jnp.float32)
        o_ref[...] = jnp.sum(xb * wb, axis=0) + b_ref[0, 0]

    return pl.pallas_call(
        body,
        grid=(_BATCH // _BLK,),
        in_specs=[
            pl.BlockSpec((_N_FEAT, _BLK), lambda i: (0, i)),
            pl.BlockSpec((_N_FEAT, 1), lambda i: (0, 0)),
            pl.BlockSpec((1, 1), lambda i: (0, 0)),
        ],
        out_specs=pl.BlockSpec((_BLK,), lambda i: (i,)),
        out_shape=jax.ShapeDtypeStruct((_BATCH,), jnp.float32),
    )(x_t, w_full_t, bias)


def _combine(parts):
    def body(*refs):
        o_ref = refs[-1]
        acc = refs[0][...]
        for r in refs[1:-1]:
            acc = acc + r[...]
        o_ref[...] = acc

    return pl.pallas_call(
        body,
        grid=(_BATCH // _BLK,),
        in_specs=[pl.BlockSpec((_BLK,), lambda i: (i,))] * len(parts),
        out_specs=pl.BlockSpec((_BLK,), lambda i: (i,)),
        out_shape=jax.ShapeDtypeStruct((_BATCH,), jnp.float32),
    )(*parts)


def kernel(x, tables, W, b):
    x_t = jnp.transpose(x, (1, 0))
    tab_t = jnp.transpose(tables, (0, 2, 1))
    w_col = W[0, : _N_CATS * _N_EMBED].reshape(_N_CATS, _N_EMBED)[:, :, None]
    w_full_t = jnp.concatenate(
        [jnp.zeros((_N_CATS,), jnp.float32), W[0, _N_CATS * _N_EMBED:]]
    ).reshape(_N_FEAT, 1)
    bias = b.reshape(1, 1)

    embs = []
    xi_flat = None
    for t0, tn in _PIECES:
        if xi_flat is None:
            p_i, xi_flat = _project_first(tab_t, w_col, x_t, t0, tn)
        else:
            p_i = _project(tab_t, w_col, t0, tn)
        embs.append(_sc_gather_sum(xi_flat, p_i, t0, tn))
    num = _numeric(x_t, w_full_t, bias)
    return _combine(embs + [num]).reshape(_BATCH, 1)


# speedup vs baseline: 1.0243x; 1.0117x over previous
import dataclasses
import functools

import jax
import jax.numpy as jnp
from jax import lax
from jax.experimental import pallas as pl
from jax.experimental.pallas import tpu as pltpu
from jax.experimental.pallas import tpu_sc as plsc

_N_CATS = 26
_VOCAB = 100000
_VPAD = 100352
_N_EMBED = 16
_N_NUM = 13
_N_FEAT = _N_CATS + _N_NUM
_BATCH = 16384

_NC = 2
_NS = 16
_NW = _NC * _NS
_SPW = _BATCH // _NW
_CHUNK = 512
_NCHUNK = _SPW // _CHUNK

_PIECES = ((0, 14), (14, 12))
_TPB = 2


def _project_first(tab_t, w_col, x_t, t0, tn):
    starts = {}
    for p0, pn in _PIECES:
        for t in range(p0, p0 + pn):
            starts[t] = p0

    def body(tab_ref, w_ref, x_ref, p_ref, xi_ref):
        for u in range(_TPB):
            p_ref[pl.ds(u * _VPAD, _VPAD)] = jnp.sum(
                tab_ref[u] * w_ref[u], axis=0)

        @pl.when(pl.program_id(0) == 0)
        def _():
            for t in range(_N_CATS):
                xi_ref[pl.ds(t * _BATCH, _BATCH)] = (
                    x_ref[t].astype(jnp.int32) + (t - starts[t]) * _VPAD)

    return pl.pallas_call(
        body,
        grid=(tn // _TPB,),
        in_specs=[
            pl.BlockSpec((_TPB, _N_EMBED, _VPAD),
                         lambda t: (t0 // _TPB + t, 0, 0)),
            pl.BlockSpec((_TPB, _N_EMBED, 1), lambda t: (t0 // _TPB + t, 0, 0)),
            pl.BlockSpec((32, _BATCH), lambda t: (0, 0)),
        ],
        out_specs=[
            pl.BlockSpec((_TPB * _VPAD,), lambda t: (t,)),
            pl.BlockSpec((_N_CATS * _BATCH,), lambda t: (0,)),
        ],
        out_shape=[
            jax.ShapeDtypeStruct((tn * _VPAD,), jnp.float32),
            jax.ShapeDtypeStruct((_N_CATS * _BATCH,), jnp.int32),
        ],
    )(tab_t, w_col, x_t)


def _project(tab_t, w_col, t0, tn):
    def body(tab_ref, w_ref, p_ref):
        for u in range(_TPB):
            p_ref[pl.ds(u * _VPAD, _VPAD)] = jnp.sum(
                tab_ref[u] * w_ref[u], axis=0)

    return pl.pallas_call(
        body,
        grid=(tn // _TPB,),
        in_specs=[
            pl.BlockSpec((_TPB, _N_EMBED, _VPAD),
                         lambda t: (t0 // _TPB + t, 0, 0)),
            pl.BlockSpec((_TPB, _N_EMBED, 1), lambda t: (t0 // _TPB + t, 0, 0)),
        ],
        out_specs=pl.BlockSpec((_TPB * _VPAD,), lambda t: (t,)),
        out_shape=jax.ShapeDtypeStruct((tn * _VPAD,), jnp.float32),
    )(tab_t, w_col)


def _sc_compiler_params():
    cp = pltpu.CompilerParams(use_tc_tiling_on_sc=False)
    if "needs_layout_passes" in pltpu.CompilerParams.__dataclass_fields__:
        cp = dataclasses.replace(cp, needs_layout_passes=False)
    return cp


def _sc_gather_sum(xi_flat, p_flat, t0, tn):
    mesh = plsc.VectorSubcoreMesh(core_axis_name="c", subcore_axis_name="s")

    @functools.partial(
        pl.kernel,
        out_type=jax.ShapeDtypeStruct((_BATCH,), jnp.float32),
        mesh=mesh,
        scratch_types=[
            pltpu.VMEM((tn * _SPW,), jnp.int32),
            pltpu.VMEM((tn * _CHUNK,), jnp.float32),
            pltpu.VMEM((_CHUNK,), jnp.float32),
            pltpu.SemaphoreType.DMA,
            pltpu.SemaphoreType.DMA,
        ],
        compiler_params=_sc_compiler_params(),
    )
    def k(xi_hbm, p_hbm, out_hbm, idx_v, vals_v, sum_v, isem, gsem):
        wid = lax.axis_index("s") * _NC + lax.axis_index("c")
        base = wid * _SPW

        idx_copies = [
            pltpu.async_copy(
                xi_hbm.at[pl.ds(
                    pl.multiple_of((t0 + j) * _BATCH + base, _SPW), _SPW)],
                idx_v.at[pl.ds(j * _SPW, _SPW)],
                isem,
            )
            for j in range(tn)
        ]
        for cp in idx_copies:
            cp.wait()

        @pl.loop(0, _NCHUNK)
        def _chunk(c):
            coff = c * _CHUNK

            gathers = [
                pltpu.async_copy(
                    p_hbm.at[idx_v.at[pl.ds(
                        pl.multiple_of(j * _SPW + coff, _CHUNK), _CHUNK)]],
                    vals_v.at[pl.ds(j * _CHUNK, _CHUNK)],
                    gsem,
                )
                for j in range(tn)
            ]
            for cp in gathers:
                cp.wait()

            for g in range(_CHUNK // 16):
                acc = vals_v[pl.ds(g * 16, 16)]
                for j in range(1, tn):
                    acc = acc + vals_v[pl.ds(j * _CHUNK + g * 16, 16)]
                sum_v[pl.ds(g * 16, 16)] = acc

            pltpu.sync_copy(
                sum_v,
                out_hbm.at[pl.ds(pl.multiple_of(base + coff, _CHUNK), _CHUNK)])

    return k(xi_flat, p_flat)


_BLK = 16384


def _numeric(x_t, w_full_t, bias):
    def body(x_ref, wn_ref, b_ref, o_ref):
        xb = x_ref[...].astype(jnp.bfloat16).astype(jnp.float32)
        wb = wn_ref[...].astype(jnp.bfloat16).astype(jnp.float32)
        o_ref[...] = jnp.sum(xb * wb, axis=0) + b_ref[0, 0]

    return pl.pallas_call(
        body,
        grid=(_BATCH // _BLK,),
        in_specs=[
            pl.BlockSpec((_N_FEAT, _BLK), lambda i: (0, i)),
            pl.BlockSpec((_N_FEAT, 1), lambda i: (0, 0)),
            pl.BlockSpec((1, 1), lambda i: (0, 0)),
        ],
        out_specs=pl.BlockSpec((_BLK,), lambda i: (i,)),
        out_shape=jax.ShapeDtypeStruct((_BATCH,), jnp.float32),
    )(x_t, w_full_t, bias)


def _combine(parts):
    def body(*refs):
        o_ref = refs[-1]
        acc = refs[0][...]
        for r in refs[1:-1]:
            acc = acc + r[...]
        o_ref[...] = acc

    return pl.pallas_call(
        body,
        grid=(_BATCH // _BLK,),
        in_specs=[pl.BlockSpec((_BLK,), lambda i: (i,))] * len(parts),
        out_specs=pl.BlockSpec((_BLK,), lambda i: (i,)),
        out_shape=jax.ShapeDtypeStruct((_BATCH,), jnp.float32),
    )(*parts)


def kernel(x, tables, W, b):
    x_t = jnp.transpose(x, (1, 0))
    tab_t = jnp.transpose(tables, (0, 2, 1))
    w_col = W[0, : _N_CATS * _N_EMBED].reshape(_N_CATS, _N_EMBED)[:, :, None]
    w_full_t = jnp.concatenate(
        [jnp.zeros((_N_CATS,), jnp.float32), W[0, _N_CATS * _N_EMBED:]]
    ).reshape(_N_FEAT, 1)
    bias = b.reshape(1, 1)

    embs = []
    xi_flat = None
    for t0, tn in _PIECES:
        if xi_flat is None:
            p_i, xi_flat = _project_first(tab_t, w_col, x_t, t0, tn)
        else:
            p_i = _project(tab_t, w_col, t0, tn)
        embs.append(_sc_gather_sum(xi_flat, p_i, t0, tn))
    num = _numeric(x_t, w_full_t, bias)
    return _combine(embs + [num]).reshape(_BATCH, 1)
